# baseline (speedup 1.0000x reference)
import functools
import math

import jax
import jax.numpy as jnp
from jax import lax
from jax.experimental import pallas as pl
from jax.experimental.pallas import tpu as pltpu
from jax.experimental.pallas import tpu_sc as plsc

_NT = 65536
_C = 32
_NCODES = 1024
_ATTN = 32
_R = 512
_SCALE = 1.0 / math.sqrt(_ATTN)


def _tc_body(hs_ref, cb_ref, wq_ref, bq_ref, wk_ref, bk_ref, wv_ref, bv_ref,
             logits_ref, idx_ref, val_ref, key_scr):
    i = pl.program_id(0)

    @pl.when(i == 0)
    def _init():
        cb = cb_ref[...]
        key_scr[...] = jnp.dot(cb, wk_ref[...]) + bk_ref[...]
        val_ref[...] = jnp.dot(cb, wv_ref[...]) + bv_ref[...]

    q = jnp.dot(hs_ref[...], wq_ref[...]) + bq_ref[...]
    logits = lax.dot_general(
        q, key_scr[...], (((1,), (1,)), ((), ()))
    ) * _SCALE
    logits_ref[...] = logits
    m = jnp.max(logits, axis=1, keepdims=True)
    iota = lax.broadcasted_iota(jnp.int32, (_R, _NCODES), 1)
    idx_ref[...] = jnp.min(
        jnp.where(logits == m, iota, _NCODES), axis=1, keepdims=True
    )


_tc_call = pl.pallas_call(
    _tc_body,
    grid=(_NT // _R,),
    in_specs=[
        pl.BlockSpec((_R, _C), lambda i: (i, 0)),
        pl.BlockSpec((_NCODES, _C), lambda i: (0, 0)),
        pl.BlockSpec((_C, _ATTN), lambda i: (0, 0)),
        pl.BlockSpec((1, _ATTN), lambda i: (0, 0)),
        pl.BlockSpec((_C, _ATTN), lambda i: (0, 0)),
        pl.BlockSpec((1, _ATTN), lambda i: (0, 0)),
        pl.BlockSpec((_C, _C), lambda i: (0, 0)),
        pl.BlockSpec((1, _C), lambda i: (0, 0)),
    ],
    out_specs=[
        pl.BlockSpec((_R, _NCODES), lambda i: (i, 0)),
        pl.BlockSpec((_R, 1), lambda i: (i, 0)),
        pl.BlockSpec((_NCODES, _C), lambda i: (0, 0)),
    ],
    out_shape=[
        jax.ShapeDtypeStruct((_NT, _NCODES), jnp.float32),
        jax.ShapeDtypeStruct((_NT, 1), jnp.int32),
        jax.ShapeDtypeStruct((_NCODES, _C), jnp.float32),
    ],
    scratch_shapes=[pltpu.VMEM((_NCODES, _ATTN), jnp.float32)],
)


_NW = 32
_BPW = _NT // _NW
_CH = 128
_NCHUNK = _BPW // _CH


@functools.lru_cache(maxsize=None)
def _make_sc_gather():
    mesh = plsc.VectorSubcoreMesh(core_axis_name="c", subcore_axis_name="s")

    @functools.partial(
        pl.kernel,
        mesh=mesh,
        compiler_params=pltpu.CompilerParams(use_tc_tiling_on_sc=False),
        out_type=(
            jax.ShapeDtypeStruct((_NT, _C), jnp.float32),
            jax.ShapeDtypeStruct((_NT, _C), jnp.float32),
        ),
        scratch_types=[
            pltpu.VMEM((_BPW,), jnp.int32),
            pltpu.VMEM((_BPW, _C), jnp.float32),
            pltpu.SemaphoreType.DMA,
        ],
    )
    def _sc_gather(idx_hbm, table_hbm, zq_hbm, zq2_hbm, idx_v, rows_v, sem):
        wid = lax.axis_index("s") * 2 + lax.axis_index("c")
        base = wid * _BPW
        pltpu.sync_copy(idx_hbm.at[pl.ds(base, _BPW)], idx_v)
        copies = []
        for j in range(_NCHUNK):
            off = j * _CH
            copies.append(pltpu.async_copy(
                table_hbm.at[idx_v.at[pl.ds(off, _CH)]],
                rows_v.at[pl.ds(off, _CH), :],
                sem,
            ))
        for c in copies:
            c.wait()
        pltpu.sync_copy(rows_v, zq_hbm.at[pl.ds(base, _BPW), :])
        pltpu.sync_copy(rows_v, zq2_hbm.at[pl.ds(base, _BPW), :])

    return _sc_gather


def kernel(hidden_states, codebook_hidden_states, Wq, bq, Wk, bk, Wv, bv):
    logits, idx, value = _tc_call(
        hidden_states, codebook_hidden_states,
        Wq, bq.reshape(1, -1), Wk, bk.reshape(1, -1), Wv, bv.reshape(1, -1),
    )
    z_q, z_q_2 = _make_sc_gather()(idx.reshape(-1), value)
    return (logits, idx, z_q, z_q_2)


# speedup vs baseline: 1.2410x; 1.2410x over previous
import functools
import math

import jax
import jax.numpy as jnp
from jax import lax
from jax.experimental import pallas as pl
from jax.experimental.pallas import tpu as pltpu
from jax.experimental.pallas import tpu_sc as plsc

_NT = 65536
_C = 32
_NCODES = 1024
_ATTN = 32
_R = 2048
_SCALE = 1.0 / math.sqrt(_ATTN)


def _tc_body(hs_ref, cb_ref, wq_ref, bq_ref, wk_ref, bk_ref, wv_ref, bv_ref,
             logits_ref, idx_ref, val_ref, key_scr):
    i = pl.program_id(0)

    @pl.when(i == 0)
    def _init():
        cb = cb_ref[...]
        key_scr[...] = jnp.dot(cb, wk_ref[...]) + bk_ref[...]
        val_ref[...] = jnp.dot(cb, wv_ref[...]) + bv_ref[...]

    q = jnp.dot(hs_ref[...], wq_ref[...]) + bq_ref[...]
    logits = lax.dot_general(
        q, key_scr[...], (((1,), (1,)), ((), ()))
    ) * _SCALE
    logits_ref[...] = logits
    m = jnp.max(logits, axis=1, keepdims=True)
    iota = lax.broadcasted_iota(jnp.int32, (_R, _NCODES), 1)
    idx_ref[...] = jnp.min(
        jnp.where(logits == m, iota, _NCODES), axis=1, keepdims=True
    )


_tc_call = pl.pallas_call(
    _tc_body,
    grid=(_NT // _R,),
    in_specs=[
        pl.BlockSpec((_R, _C), lambda i: (i, 0)),
        pl.BlockSpec((_NCODES, _C), lambda i: (0, 0)),
        pl.BlockSpec((_C, _ATTN), lambda i: (0, 0)),
        pl.BlockSpec((1, _ATTN), lambda i: (0, 0)),
        pl.BlockSpec((_C, _ATTN), lambda i: (0, 0)),
        pl.BlockSpec((1, _ATTN), lambda i: (0, 0)),
        pl.BlockSpec((_C, _C), lambda i: (0, 0)),
        pl.BlockSpec((1, _C), lambda i: (0, 0)),
    ],
    out_specs=[
        pl.BlockSpec((_R, _NCODES), lambda i: (i, 0)),
        pl.BlockSpec((_R, 1), lambda i: (i, 0)),
        pl.BlockSpec((_NCODES, _C), lambda i: (0, 0)),
    ],
    out_shape=[
        jax.ShapeDtypeStruct((_NT, _NCODES), jnp.float32),
        jax.ShapeDtypeStruct((_NT, 1), jnp.int32),
        jax.ShapeDtypeStruct((_NCODES, _C), jnp.float32),
    ],
    scratch_shapes=[pltpu.VMEM((_NCODES, _ATTN), jnp.float32)],
)


_NW = 32
_BPW = _NT // _NW
_CH = 128
_NCHUNK = _BPW // _CH


@functools.lru_cache(maxsize=None)
def _make_sc_gather():
    mesh = plsc.VectorSubcoreMesh(core_axis_name="c", subcore_axis_name="s")

    @functools.partial(
        pl.kernel,
        mesh=mesh,
        compiler_params=pltpu.CompilerParams(use_tc_tiling_on_sc=False),
        out_type=(
            jax.ShapeDtypeStruct((_NT, _C), jnp.float32),
            jax.ShapeDtypeStruct((_NT, _C), jnp.float32),
        ),
        scratch_types=[
            pltpu.VMEM((_BPW,), jnp.int32),
            pltpu.VMEM((_BPW, _C), jnp.float32),
            pltpu.SemaphoreType.DMA,
        ],
    )
    def _sc_gather(idx_hbm, table_hbm, zq_hbm, zq2_hbm, idx_v, rows_v, sem):
        wid = lax.axis_index("s") * 2 + lax.axis_index("c")
        base = wid * _BPW
        pltpu.sync_copy(idx_hbm.at[pl.ds(base, _BPW)], idx_v)
        copies = []
        for j in range(_NCHUNK):
            off = j * _CH
            copies.append(pltpu.async_copy(
                table_hbm.at[idx_v.at[pl.ds(off, _CH)]],
                rows_v.at[pl.ds(off, _CH), :],
                sem,
            ))
        for c in copies:
            c.wait()
        pltpu.sync_copy(rows_v, zq_hbm.at[pl.ds(base, _BPW), :])
        pltpu.sync_copy(rows_v, zq2_hbm.at[pl.ds(base, _BPW), :])

    return _sc_gather


def kernel(hidden_states, codebook_hidden_states, Wq, bq, Wk, bk, Wv, bv):
    logits, idx, value = _tc_call(
        hidden_states, codebook_hidden_states,
        Wq, bq.reshape(1, -1), Wk, bk.reshape(1, -1), Wv, bv.reshape(1, -1),
    )
    z_q, z_q_2 = _make_sc_gather()(idx.reshape(-1), value)
    return (logits, idx, z_q, z_q_2)


# speedup vs baseline: 1.2460x; 1.0040x over previous
import functools
import math

import jax
import jax.numpy as jnp
from jax import lax
from jax.experimental import pallas as pl
from jax.experimental.pallas import tpu as pltpu
from jax.experimental.pallas import tpu_sc as plsc

_NT = 65536
_C = 32
_NCODES = 1024
_ATTN = 32
_R = 4096
_SCALE = 1.0 / math.sqrt(_ATTN)


def _tc_body(hs_ref, cb_ref, wq_ref, bq_ref, wk_ref, bk_ref, wv_ref, bv_ref,
             logits_ref, idx_ref, val_ref, key_scr):
    i = pl.program_id(0)

    @pl.when(i == 0)
    def _init():
        cb = cb_ref[...]
        key_scr[...] = jnp.dot(cb, wk_ref[...]) + bk_ref[...]
        val_ref[...] = jnp.dot(cb, wv_ref[...]) + bv_ref[...]

    q = jnp.dot(hs_ref[...], wq_ref[...]) + bq_ref[...]
    logits = lax.dot_general(
        q, key_scr[...], (((1,), (1,)), ((), ()))
    ) * _SCALE
    logits_ref[...] = logits
    m = jnp.max(logits, axis=1, keepdims=True)
    iota = lax.broadcasted_iota(jnp.int32, (_R, _NCODES), 1)
    idx_ref[...] = jnp.min(
        jnp.where(logits == m, iota, _NCODES), axis=1, keepdims=True
    )


_tc_call = pl.pallas_call(
    _tc_body,
    grid=(_NT // _R,),
    in_specs=[
        pl.BlockSpec((_R, _C), lambda i: (i, 0)),
        pl.BlockSpec((_NCODES, _C), lambda i: (0, 0)),
        pl.BlockSpec((_C, _ATTN), lambda i: (0, 0)),
        pl.BlockSpec((1, _ATTN), lambda i: (0, 0)),
        pl.BlockSpec((_C, _ATTN), lambda i: (0, 0)),
        pl.BlockSpec((1, _ATTN), lambda i: (0, 0)),
        pl.BlockSpec((_C, _C), lambda i: (0, 0)),
        pl.BlockSpec((1, _C), lambda i: (0, 0)),
    ],
    out_specs=[
        pl.BlockSpec((_R, _NCODES), lambda i: (i, 0)),
        pl.BlockSpec((_R, 1), lambda i: (i, 0)),
        pl.BlockSpec((_NCODES, _C), lambda i: (0, 0)),
    ],
    out_shape=[
        jax.ShapeDtypeStruct((_NT, _NCODES), jnp.float32),
        jax.ShapeDtypeStruct((_NT, 1), jnp.int32),
        jax.ShapeDtypeStruct((_NCODES, _C), jnp.float32),
    ],
    scratch_shapes=[pltpu.VMEM((_NCODES, _ATTN), jnp.float32)],
)


_NW = 32
_BPW = _NT // _NW
_CH = 128
_NCHUNK = _BPW // _CH


@functools.lru_cache(maxsize=None)
def _make_sc_gather():
    mesh = plsc.VectorSubcoreMesh(core_axis_name="c", subcore_axis_name="s")

    @functools.partial(
        pl.kernel,
        mesh=mesh,
        compiler_params=pltpu.CompilerParams(use_tc_tiling_on_sc=False),
        out_type=(
            jax.ShapeDtypeStruct((_NT, _C), jnp.float32),
            jax.ShapeDtypeStruct((_NT, _C), jnp.float32),
        ),
        scratch_types=[
            pltpu.VMEM((_BPW,), jnp.int32),
            pltpu.VMEM((_BPW, _C), jnp.float32),
            pltpu.SemaphoreType.DMA,
        ],
    )
    def _sc_gather(idx_hbm, table_hbm, zq_hbm, zq2_hbm, idx_v, rows_v, sem):
        wid = lax.axis_index("s") * 2 + lax.axis_index("c")
        base = wid * _BPW
        pltpu.sync_copy(idx_hbm.at[pl.ds(base, _BPW)], idx_v)
        copies = []
        for j in range(_NCHUNK):
            off = j * _CH
            copies.append(pltpu.async_copy(
                table_hbm.at[idx_v.at[pl.ds(off, _CH)]],
                rows_v.at[pl.ds(off, _CH), :],
                sem,
            ))
        for c in copies:
            c.wait()
        pltpu.sync_copy(rows_v, zq_hbm.at[pl.ds(base, _BPW), :])
        pltpu.sync_copy(rows_v, zq2_hbm.at[pl.ds(base, _BPW), :])

    return _sc_gather


def kernel(hidden_states, codebook_hidden_states, Wq, bq, Wk, bk, Wv, bv):
    logits, idx, value = _tc_call(
        hidden_states, codebook_hidden_states,
        Wq, bq.reshape(1, -1), Wk, bk.reshape(1, -1), Wv, bv.reshape(1, -1),
    )
    z_q, z_q_2 = _make_sc_gather()(idx.reshape(-1), value)
    return (logits, idx, z_q, z_q_2)
